# baseline (speedup 1.0000x reference)
import functools

import jax
import jax.numpy as jnp
from jax import lax
from jax.experimental import pallas as pl
from jax.experimental.pallas import tpu as pltpu
from jax.experimental.pallas import tpu_sc as plsc

BATCH = 16384
VOCAB = 1000
EMBED = 64
NC = 2
NS = 16
NW = NC * NS
BPW = BATCH // NW
CHUNK = 512
NCH = BPW // CHUNK


def _sc_embed(idx0, idx1, table):
  mesh = plsc.VectorSubcoreMesh(core_axis_name="c", subcore_axis_name="s")

  @functools.partial(
      pl.kernel,
      mesh=mesh,
      compiler_params=pltpu.CompilerParams(use_tc_tiling_on_sc=False,
                                           needs_layout_passes=False,
                                           disable_bounds_checks=True,
                                           disable_semaphore_checks=True),
      out_type=jax.ShapeDtypeStruct((BATCH, 2 * EMBED), jnp.float32),
      scratch_types=[
          pltpu.VMEM((BPW,), jnp.int32),
          pltpu.VMEM((BPW,), jnp.int32),
          pltpu.VMEM((BPW, EMBED), jnp.float32),
          pltpu.VMEM((BPW, EMBED), jnp.float32),
          pltpu.VMEM_SHARED((2 * VOCAB, EMBED), jnp.float32),
          pltpu.SemaphoreType.DMA,
          pltpu.SemaphoreType.DMA,
          pltpu.SemaphoreType.DMA,
      ],
  )
  def body(idx0_hbm, idx1_hbm, tbl_hbm, out_hbm,
           i0_v, i1_v, r0_v, r1_v, tbl_s, isem, gsem, wsem):
    sid = lax.axis_index("s")
    wid = sid * NC + lax.axis_index("c")
    base = wid * BPW
    l0 = pltpu.async_copy(idx0_hbm.at[pl.ds(base, BPW)], i0_v, isem)
    l1 = pltpu.async_copy(idx1_hbm.at[pl.ds(base, BPW)], i1_v, isem)
    rpt = 2 * VOCAB // NS
    pltpu.sync_copy(tbl_hbm.at[pl.ds(sid * rpt, rpt)],
                    tbl_s.at[pl.ds(sid * rpt, rpt)])
    plsc.subcore_barrier()
    l0.wait()
    g0 = pltpu.async_copy(tbl_s.at[i0_v], r0_v, gsem)
    l1.wait()
    g1 = pltpu.async_copy(tbl_s.at[i1_v], r1_v, gsem)
    g0.wait()
    g1.wait()
    w0 = pltpu.async_copy(
        r0_v, out_hbm.at[pl.ds(base, BPW), pl.ds(0, EMBED)], wsem)
    w1 = pltpu.async_copy(
        r1_v, out_hbm.at[pl.ds(base, BPW), pl.ds(EMBED, EMBED)], wsem)
    w0.wait()
    w1.wait()

  return body(idx0, idx1, table)


def kernel(symbolic_instructions_batch, landmark_table, theta_table,
           radius_table):
  sib = symbolic_instructions_batch.astype(jnp.int32)
  table = jnp.concatenate([landmark_table, theta_table], axis=0)
  return _sc_embed(sib[:, 0], sib[:, 1] + VOCAB, table)


# speedup vs baseline: 1.0037x; 1.0037x over previous
import functools

import jax
import jax.numpy as jnp
from jax import lax
from jax.experimental import pallas as pl
from jax.experimental.pallas import tpu as pltpu
from jax.experimental.pallas import tpu_sc as plsc

BATCH = 16384
VOCAB = 1000
EMBED = 64
NC = 2
NS = 16
NW = NC * NS
BPW = BATCH // NW


def _sc_embed(idx0, idx1, table):
  mesh = plsc.VectorSubcoreMesh(core_axis_name="c", subcore_axis_name="s")

  @functools.partial(
      pl.kernel,
      mesh=mesh,
      compiler_params=pltpu.CompilerParams(use_tc_tiling_on_sc=False,
                                           needs_layout_passes=False,
                                           disable_bounds_checks=True,
                                           disable_semaphore_checks=True),
      out_type=jax.ShapeDtypeStruct((BATCH, 2 * EMBED), jnp.float32),
      scratch_types=[
          pltpu.VMEM((BPW,), jnp.int32),
          pltpu.VMEM((BPW,), jnp.int32),
          pltpu.VMEM((BPW, EMBED), jnp.float32),
          pltpu.VMEM((BPW, EMBED), jnp.float32),
          pltpu.VMEM_SHARED((2 * VOCAB, EMBED), jnp.float32),
          pltpu.SemaphoreType.DMA,
          pltpu.SemaphoreType.DMA,
          pltpu.SemaphoreType.DMA,
      ],
  )
  def body(idx0_hbm, idx1_hbm, tbl_hbm, out_hbm,
           i0_v, i1_v, r0_v, r1_v, tbl_s, isem, gsem, wsem):
    sid = lax.axis_index("s")
    wid = sid * NC + lax.axis_index("c")
    base = wid * BPW
    l0 = pltpu.async_copy(idx0_hbm.at[pl.ds(base, BPW)], i0_v, isem)
    l1 = pltpu.async_copy(idx1_hbm.at[pl.ds(base, BPW)], i1_v, isem)
    rpt = 2 * VOCAB // NS
    pltpu.sync_copy(tbl_hbm.at[pl.ds(sid * rpt, rpt)],
                    tbl_s.at[pl.ds(sid * rpt, rpt)])
    plsc.subcore_barrier()
    l0.wait()
    g0 = pltpu.async_copy(tbl_s.at[i0_v], r0_v, gsem)
    l1.wait()
    g1 = pltpu.async_copy(tbl_s.at[i1_v], r1_v, gsem)
    g0.wait()
    g1.wait()
    w0 = pltpu.async_copy(
        r0_v, out_hbm.at[pl.ds(base, BPW), pl.ds(0, EMBED)], wsem)
    w1 = pltpu.async_copy(
        r1_v, out_hbm.at[pl.ds(base, BPW), pl.ds(EMBED, EMBED)], wsem)
    w0.wait()
    w1.wait()

  return body(idx0, idx1, table)


def kernel(symbolic_instructions_batch, landmark_table, theta_table,
           radius_table):
  sib = symbolic_instructions_batch.astype(jnp.int32)
  table = jnp.concatenate([landmark_table, theta_table], axis=0)
  return _sc_embed(sib[:, 0], sib[:, 1] + VOCAB, table)
